# baseline (speedup 1.0000x reference)
import jax
import jax.numpy as jnp
from jax import lax
from jax.experimental import pallas as pl
from jax.experimental.pallas import tpu as pltpu

_B = 128
_N = 8192
_NCH = 8
_CH = _B // _NCH


def _body(x_hbm, o_hbm, xv, ov, rsem, wsem):
    iota = lax.broadcasted_iota(jnp.int32, (_CH, _N), 1)
    in_cp = []
    for c in range(_NCH):
        cp = pltpu.make_async_copy(
            x_hbm.at[pl.ds(c * _CH, _CH)], xv.at[pl.ds(c * _CH, _CH)],
            rsem.at[c],
        )
        cp.start()
        in_cp.append(cp)
    out_cp = []
    for c in range(_NCH):
        in_cp[c].wait()
        x = xv[pl.ds(c * _CH, _CH), :]
        m = jnp.max(x, axis=1, keepdims=True)
        cand = jnp.where(x == m, iota, _N)
        idx = jnp.min(cand, axis=1, keepdims=True)
        ov[pl.ds(c * _CH, _CH), :] = (iota == idx).astype(jnp.float32)
        cp = pltpu.make_async_copy(
            ov.at[pl.ds(c * _CH, _CH)], o_hbm.at[pl.ds(c * _CH, _CH)],
            wsem.at[c],
        )
        cp.start()
        out_cp.append(cp)
    for cp in out_cp:
        cp.wait()


def kernel(coords):
    return pl.pallas_call(
        _body,
        out_shape=jax.ShapeDtypeStruct((_B, _N), jnp.float32),
        in_specs=[pl.BlockSpec(memory_space=pl.ANY)],
        out_specs=pl.BlockSpec(memory_space=pl.ANY),
        scratch_shapes=[
            pltpu.VMEM((_B, _N), jnp.float32),
            pltpu.VMEM((_B, _N), jnp.float32),
            pltpu.SemaphoreType.DMA((_NCH,)),
            pltpu.SemaphoreType.DMA((_NCH,)),
        ],
    )(coords)


# speedup vs baseline: 1.1144x; 1.1144x over previous
import jax
import jax.numpy as jnp
from jax import lax
from jax.experimental import pallas as pl
from jax.experimental.pallas import tpu as pltpu

_B = 128
_N = 8192
_NCH = 2
_CH = _B // _NCH


def _body(x_hbm, o_hbm, xv, ov, rsem, wsem):
    iota = lax.broadcasted_iota(jnp.int32, (_CH, _N), 1)
    in_cp = []
    for c in range(_NCH):
        cp = pltpu.make_async_copy(
            x_hbm.at[pl.ds(c * _CH, _CH)], xv.at[pl.ds(c * _CH, _CH)],
            rsem.at[c],
        )
        cp.start()
        in_cp.append(cp)
    out_cp = []
    for c in range(_NCH):
        in_cp[c].wait()
        x = xv[pl.ds(c * _CH, _CH), :]
        m = jnp.max(x, axis=1, keepdims=True)
        cand = jnp.where(x == m, iota, _N)
        idx = jnp.min(cand, axis=1, keepdims=True)
        ov[pl.ds(c * _CH, _CH), :] = (iota == idx).astype(jnp.float32)
        cp = pltpu.make_async_copy(
            ov.at[pl.ds(c * _CH, _CH)], o_hbm.at[pl.ds(c * _CH, _CH)],
            wsem.at[c],
        )
        cp.start()
        out_cp.append(cp)
    for cp in out_cp:
        cp.wait()


def kernel(coords):
    return pl.pallas_call(
        _body,
        out_shape=jax.ShapeDtypeStruct((_B, _N), jnp.float32),
        in_specs=[pl.BlockSpec(memory_space=pl.ANY)],
        out_specs=pl.BlockSpec(memory_space=pl.ANY),
        scratch_shapes=[
            pltpu.VMEM((_B, _N), jnp.float32),
            pltpu.VMEM((_B, _N), jnp.float32),
            pltpu.SemaphoreType.DMA((_NCH,)),
            pltpu.SemaphoreType.DMA((_NCH,)),
        ],
    )(coords)


# speedup vs baseline: 1.1741x; 1.0536x over previous
import jax
import jax.numpy as jnp
from jax import lax
from jax.experimental import pallas as pl
from jax.experimental.pallas import tpu as pltpu

_B = 128
_N = 8192
_SIZES = (16, 48, 48, 16)
_OFFS = (0, 16, 64, 112)
_NCH = len(_SIZES)


def _body(x_hbm, o_hbm, xv, ov, rsem, wsem):
    in_cp = []
    for c in range(_NCH):
        cp = pltpu.make_async_copy(
            x_hbm.at[pl.ds(_OFFS[c], _SIZES[c])],
            xv.at[pl.ds(_OFFS[c], _SIZES[c])],
            rsem.at[c],
        )
        cp.start()
        in_cp.append(cp)
    out_cp = []
    for c in range(_NCH):
        in_cp[c].wait()
        x = xv[pl.ds(_OFFS[c], _SIZES[c]), :]
        iota = lax.broadcasted_iota(jnp.int32, (_SIZES[c], _N), 1)
        m = jnp.max(x, axis=1, keepdims=True)
        cand = jnp.where(x == m, iota, _N)
        idx = jnp.min(cand, axis=1, keepdims=True)
        ov[pl.ds(_OFFS[c], _SIZES[c]), :] = (iota == idx).astype(jnp.float32)
        cp = pltpu.make_async_copy(
            ov.at[pl.ds(_OFFS[c], _SIZES[c])],
            o_hbm.at[pl.ds(_OFFS[c], _SIZES[c])],
            wsem.at[c],
        )
        cp.start()
        out_cp.append(cp)
    for cp in out_cp:
        cp.wait()


def kernel(coords):
    return pl.pallas_call(
        _body,
        out_shape=jax.ShapeDtypeStruct((_B, _N), jnp.float32),
        in_specs=[pl.BlockSpec(memory_space=pl.ANY)],
        out_specs=pl.BlockSpec(memory_space=pl.ANY),
        scratch_shapes=[
            pltpu.VMEM((_B, _N), jnp.float32),
            pltpu.VMEM((_B, _N), jnp.float32),
            pltpu.SemaphoreType.DMA((_NCH,)),
            pltpu.SemaphoreType.DMA((_NCH,)),
        ],
    )(coords)


# speedup vs baseline: 1.1891x; 1.0128x over previous
import jax
import jax.numpy as jnp
from jax import lax
from jax.experimental import pallas as pl
from jax.experimental.pallas import tpu as pltpu

_B = 128
_N = 8192
_SIZES = (8, 40, 48, 24, 8)
_OFFS = (0, 8, 48, 96, 120)
_NCH = len(_SIZES)


def _body(x_hbm, o_hbm, xv, ov, rsem, wsem):
    in_cp = []
    for c in range(_NCH):
        cp = pltpu.make_async_copy(
            x_hbm.at[pl.ds(_OFFS[c], _SIZES[c])],
            xv.at[pl.ds(_OFFS[c], _SIZES[c])],
            rsem.at[c],
        )
        cp.start()
        in_cp.append(cp)
    out_cp = []
    for c in range(_NCH):
        in_cp[c].wait()
        x = xv[pl.ds(_OFFS[c], _SIZES[c]), :]
        iota = lax.broadcasted_iota(jnp.int32, (_SIZES[c], _N), 1)
        m = jnp.max(x, axis=1, keepdims=True)
        cand = jnp.where(x == m, iota, _N)
        idx = jnp.min(cand, axis=1, keepdims=True)
        ov[pl.ds(_OFFS[c], _SIZES[c]), :] = (iota == idx).astype(jnp.float32)
        cp = pltpu.make_async_copy(
            ov.at[pl.ds(_OFFS[c], _SIZES[c])],
            o_hbm.at[pl.ds(_OFFS[c], _SIZES[c])],
            wsem.at[c],
        )
        cp.start()
        out_cp.append(cp)
    for cp in out_cp:
        cp.wait()


def kernel(coords):
    return pl.pallas_call(
        _body,
        out_shape=jax.ShapeDtypeStruct((_B, _N), jnp.float32),
        in_specs=[pl.BlockSpec(memory_space=pl.ANY)],
        out_specs=pl.BlockSpec(memory_space=pl.ANY),
        scratch_shapes=[
            pltpu.VMEM((_B, _N), jnp.float32),
            pltpu.VMEM((_B, _N), jnp.float32),
            pltpu.SemaphoreType.DMA((_NCH,)),
            pltpu.SemaphoreType.DMA((_NCH,)),
        ],
    )(coords)
